# Initial kernel scaffold; baseline (speedup 1.0000x reference)
import jax
import jax.numpy as jnp
from jax.experimental import pallas as pl


def kernel(new_xyz, xyz, points):
    raise NotImplementedError("write your pallas kernel here")


# speedup vs baseline: 10.9050x; 10.9050x over previous
import jax
import jax.numpy as jnp
from jax import lax
from jax.experimental import pallas as pl
from jax.experimental.pallas import tpu as pltpu
from jax.experimental.pallas import tpu_sc as plsc

_B, _P, _N, _S, _C = 8, 1024, 4096, 32, 64
_D = 3 + _C
_DP = 80
_L = 16
_NC, _NS = 2, 16
_NW = _NC * _NS
_PC = (_B * _P) // _NW
_QW = _P // _PC
_G = 4
_GR = _G * _S
_NCHUNK = _N // _L
_T2 = float.fromhex("0x1.47ae14p-5")
_INF = float("inf")
_BIGI = 2 ** 30


def _sc_body(cxt, xyzt, aug, npts, idxo,
             xv, yv, zv, ccx, ccy, ccz, cd, ci, idxg, gidx, rows, sem):
    wid = lax.axis_index("s") * _NC + lax.axis_index("c")
    b = wid // _QW
    cb = (wid % _QW) * _PC

    pltpu.sync_copy(xyzt.at[pl.ds((b * 3 + 0) * _N, _N)], xv)
    pltpu.sync_copy(xyzt.at[pl.ds((b * 3 + 1) * _N, _N)], yv)
    pltpu.sync_copy(xyzt.at[pl.ds((b * 3 + 2) * _N, _N)], zv)
    pltpu.sync_copy(cxt.at[pl.ds((b * 3 + 0) * _P + cb, _PC)], ccx)
    pltpu.sync_copy(cxt.at[pl.ds((b * 3 + 1) * _P + cb, _PC)], ccy)
    pltpu.sync_copy(cxt.at[pl.ds((b * 3 + 2) * _P + cb, _PC)], ccz)

    lane = lax.broadcasted_iota(jnp.int32, (_L,), 0)
    inf16 = jnp.full((_L,), _INF, jnp.float32)
    zero16i = jnp.zeros((_L,), jnp.int32)

    def centre(i, carry):
        isplat = jnp.full((_L,), i, jnp.int32)
        cx = plsc.load_gather(ccx, [isplat])
        cy = plsc.load_gather(ccy, [isplat])
        cz = plsc.load_gather(ccz, [isplat])

        def chunk(j, off):
            base = j * _L
            dx = xv[pl.ds(base, _L)] - cx
            dy = yv[pl.ds(base, _L)] - cy
            dz = zv[pl.ds(base, _L)] - cz
            d2 = dx * dx + dy * dy + dz * dz
            msk = d2 < _T2
            plsc.store_compressed(cd.at[pl.ds(off, _L)], d2, mask=msk)
            plsc.store_compressed(ci.at[pl.ds(off, _L)], lane + base, mask=msk)
            return off + jnp.sum(msk.astype(jnp.int32))

        mcount = lax.fori_loop(0, _NCHUNK, chunk, jnp.int32(0))

        def no_cand(_):
            def amin_chunk(j, st):
                rmin, ridx = st
                base = j * _L
                dx = xv[pl.ds(base, _L)] - cx
                dy = yv[pl.ds(base, _L)] - cy
                dz = zv[pl.ds(base, _L)] - cz
                d2 = dx * dx + dy * dy + dz * dz
                upd = d2 < rmin
                return jnp.where(upd, d2, rmin), jnp.where(upd, lane + base, ridx)

            rmin, ridx = lax.fori_loop(0, _NCHUNK, amin_chunk, (inf16, zero16i))
            mval = jnp.min(rmin)
            am = jnp.min(jnp.where(rmin == mval, ridx, jnp.int32(_BIGI)))
            cd[pl.ds(0, _L)] = jnp.where(lane == 0, jnp.float32(0.0), inf16)
            ci[pl.ds(0, _L)] = jnp.full((_L,), am, jnp.int32)
            return jnp.int32(1)

        def have_cand(_):
            cd[pl.ds(mcount, _L)] = inf16
            return mcount

        m = lax.cond(mcount == 0, no_cand, have_cand, 0)
        kk = jnp.minimum(m, _S)
        nc = (m + _L - 1) // _L
        gl = (i % _G) * _S

        def extract(k, carry2):
            def scanc(j, st):
                rmin, rpos = st
                cvec = cd[pl.ds(j * _L, _L)]
                upd = cvec < rmin
                return jnp.where(upd, cvec, rmin), jnp.where(upd, j, rpos)

            rmin, rpos = lax.fori_loop(0, nc, scanc, (inf16, zero16i))
            mval = jnp.min(rmin)
            pos = jnp.min(jnp.where(rmin == mval, rpos * _L + lane,
                                    jnp.int32(_BIGI)))
            psplat = jnp.full((_L,), pos, jnp.int32)
            cidx = plsc.load_gather(ci, [psplat])

            @pl.when(k == 0)
            def _():
                idxg[pl.ds(gl, _L)] = cidx
                idxg[pl.ds(gl + _L, _L)] = cidx

            plsc.store_scatter(idxg, [jnp.full((_L,), gl + k, jnp.int32)],
                               cidx, mask=lane == 0)
            plsc.store_scatter(cd, [psplat], inf16, mask=lane == 0)
            return carry2

        lax.fori_loop(0, kk, extract, jnp.int32(0))

        @pl.when(i % _G == _G - 1)
        def _():
            boff = jnp.full((_L,), b * _N, jnp.int32)
            for t in range(_GR // _L):
                gidx[pl.ds(t * _L, _L)] = idxg[pl.ds(t * _L, _L)] + boff
            rowbase = (b * _P + cb + (i - (_G - 1))) * _S
            pltpu.sync_copy(idxg, idxo.at[pl.ds(rowbase, _GR)])
            pltpu.async_copy(aug.at[gidx], rows, sem).wait()
            pltpu.sync_copy(rows, npts.at[pl.ds(rowbase, _GR)])

        return carry

    lax.fori_loop(0, _PC, centre, jnp.int32(0))


def kernel(new_xyz, xyz, points):
    cxt = jnp.transpose(new_xyz, (0, 2, 1)).reshape(-1)
    xyzt = jnp.transpose(xyz, (0, 2, 1)).reshape(-1)
    pad = jnp.zeros((_B, _N, _DP - _D), jnp.float32)
    aug = jnp.concatenate([xyz, points, pad], axis=-1).reshape(_B * _N, _DP)

    mesh = plsc.VectorSubcoreMesh(core_axis_name="c", subcore_axis_name="s",
                                  num_cores=_NC, num_subcores=_NS)
    out_type = (
        jax.ShapeDtypeStruct((_B * _P * _S, _DP), jnp.float32),
        jax.ShapeDtypeStruct((_B * _P * _S,), jnp.int32),
    )
    scratch = [
        pltpu.VMEM((_N,), jnp.float32),
        pltpu.VMEM((_N,), jnp.float32),
        pltpu.VMEM((_N,), jnp.float32),
        pltpu.VMEM((_PC,), jnp.float32),
        pltpu.VMEM((_PC,), jnp.float32),
        pltpu.VMEM((_PC,), jnp.float32),
        pltpu.VMEM((_N + _L,), jnp.float32),
        pltpu.VMEM((_N + _L,), jnp.int32),
        pltpu.VMEM((_GR,), jnp.int32),
        pltpu.VMEM((_GR,), jnp.int32),
        pltpu.VMEM((_GR, _DP), jnp.float32),
        pltpu.SemaphoreType.DMA,
    ]
    f = pl.kernel(_sc_body, out_type=out_type, mesh=mesh, scratch_types=scratch,
                  compiler_params=pltpu.CompilerParams(
                      needs_layout_passes=False, use_tc_tiling_on_sc=False))
    nppad, idxf = f(cxt, xyzt, aug)
    nppad = nppad.reshape(_B, _P, _S, _DP)
    return (nppad[..., :_D], idxf.reshape(_B, _P, _S), nppad[..., :3])


# speedup vs baseline: 12.5285x; 1.1489x over previous
import jax
import jax.numpy as jnp
from jax import lax
from jax.experimental import pallas as pl
from jax.experimental.pallas import tpu as pltpu
from jax.experimental.pallas import tpu_sc as plsc

_B, _P, _N, _S, _C = 8, 1024, 4096, 32, 64
_D = 3 + _C
_DP = 80
_L = 16
_NC, _NS = 2, 16
_NW = _NC * _NS
_PC = (_B * _P) // _NW
_QW = _P // _PC
_G = 4
_GR = _G * _S
_NCHUNK = _N // _L
_T2 = float.fromhex("0x1.47ae14p-5")
_INF = float("inf")
_BIGI = 2 ** 30


def _sc_body(cxt, xyzt, aug, npts, idxo,
             xv, yv, zv, ccx, ccy, ccz, cd, ci, idxg, gidx, rows, sem):
    wid = lax.axis_index("s") * _NC + lax.axis_index("c")
    b = wid // _QW
    cb = (wid % _QW) * _PC

    pltpu.sync_copy(xyzt.at[pl.ds((b * 3 + 0) * _N, _N)], xv)
    pltpu.sync_copy(xyzt.at[pl.ds((b * 3 + 1) * _N, _N)], yv)
    pltpu.sync_copy(xyzt.at[pl.ds((b * 3 + 2) * _N, _N)], zv)
    pltpu.sync_copy(cxt.at[pl.ds((b * 3 + 0) * _P + cb, _PC)], ccx)
    pltpu.sync_copy(cxt.at[pl.ds((b * 3 + 1) * _P + cb, _PC)], ccy)
    pltpu.sync_copy(cxt.at[pl.ds((b * 3 + 2) * _P + cb, _PC)], ccz)

    lane = lax.broadcasted_iota(jnp.int32, (_L,), 0)
    inf16 = jnp.full((_L,), _INF, jnp.float32)
    zero16i = jnp.zeros((_L,), jnp.int32)

    def centre(i, carry):
        isplat = jnp.full((_L,), i, jnp.int32)
        cx = plsc.load_gather(ccx, [isplat])
        cy = plsc.load_gather(ccy, [isplat])
        cz = plsc.load_gather(ccz, [isplat])

        def chunk(j, offv):
            base = j * _L
            dx = xv[pl.ds(base, _L)] - cx
            dy = yv[pl.ds(base, _L)] - cy
            dz = zv[pl.ds(base, _L)] - cz
            d2 = dx * dx + dy * dy + dz * dz
            msk = d2 < _T2
            pos = offv + plsc.cumsum(msk.astype(jnp.int32)) - 1
            plsc.store_scatter(cd, [pos], d2, mask=msk)
            plsc.store_scatter(ci, [pos], lane + base, mask=msk)
            return offv + plsc.all_reduce_population_count(msk)

        offv = lax.fori_loop(0, _NCHUNK, chunk, zero16i)
        mcount = jnp.sum(jnp.where(lane == 0, offv, 0))

        def no_cand(_):
            def amin_chunk(j, st):
                rmin, ridx = st
                base = j * _L
                dx = xv[pl.ds(base, _L)] - cx
                dy = yv[pl.ds(base, _L)] - cy
                dz = zv[pl.ds(base, _L)] - cz
                d2 = dx * dx + dy * dy + dz * dz
                upd = d2 < rmin
                return jnp.where(upd, d2, rmin), jnp.where(upd, lane + base, ridx)

            rmin, ridx = lax.fori_loop(0, _NCHUNK, amin_chunk, (inf16, zero16i))
            mval = jnp.min(rmin)
            am = jnp.min(jnp.where(rmin == mval, ridx, jnp.int32(_BIGI)))
            cd[pl.ds(0, _L)] = jnp.where(lane == 0, jnp.float32(0.0), inf16)
            ci[pl.ds(0, _L)] = jnp.full((_L,), am, jnp.int32)
            return jnp.int32(1)

        def have_cand(_):
            cd[pl.ds(mcount, _L)] = inf16
            return mcount

        m = lax.cond(mcount == 0, no_cand, have_cand, 0)
        nc = (m + _L - 1) // _L
        gl = (i % _G) * _S

        def merge(c, st):
            k0, v0, k1, v1 = st
            dk, dv = plsc.sort_key_val(cd[pl.ds(c * _L, _L)],
                                       ci[pl.ds(c * _L, _L)],
                                       descending=True)
            m1 = k1 <= dk
            tk = jnp.where(m1, k1, dk)
            tv = jnp.where(m1, v1, dv)
            tk, tv = plsc.sort_key_val(tk, tv, descending=True)
            m2 = k0 <= tk
            lok = jnp.where(m2, k0, tk)
            lov = jnp.where(m2, v0, tv)
            hik = jnp.where(m2, tk, k0)
            hiv = jnp.where(m2, tv, v0)
            k0n, v0n = plsc.sort_key_val(lok, lov)
            k1n, v1n = plsc.sort_key_val(hik, hiv)
            return k0n, v0n, k1n, v1n

        k0, v0, k1, v1 = lax.fori_loop(0, nc, merge,
                                       (inf16, zero16i, inf16, zero16i))
        c0 = jnp.sum(jnp.where(lane == 0, v0, 0))
        c0v = jnp.full((_L,), c0, jnp.int32)
        idxg[pl.ds(gl, _L)] = jnp.where(k0 == _INF, c0v, v0)
        idxg[pl.ds(gl + _L, _L)] = jnp.where(k1 == _INF, c0v, v1)

        @pl.when(i % _G == _G - 1)
        def _():
            boff = jnp.full((_L,), b * _N, jnp.int32)
            for t in range(_GR // _L):
                gidx[pl.ds(t * _L, _L)] = idxg[pl.ds(t * _L, _L)] + boff
            rowbase = (b * _P + cb + (i - (_G - 1))) * _S
            pltpu.sync_copy(idxg, idxo.at[pl.ds(rowbase, _GR)])
            pltpu.async_copy(aug.at[gidx], rows, sem).wait()
            pltpu.sync_copy(rows, npts.at[pl.ds(rowbase, _GR)])

        return carry

    lax.fori_loop(0, _PC, centre, jnp.int32(0))


def kernel(new_xyz, xyz, points):
    cxt = jnp.transpose(new_xyz, (0, 2, 1)).reshape(-1)
    xyzt = jnp.transpose(xyz, (0, 2, 1)).reshape(-1)
    pad = jnp.zeros((_B, _N, _DP - _D), jnp.float32)
    aug = jnp.concatenate([xyz, points, pad], axis=-1).reshape(_B * _N, _DP)

    mesh = plsc.VectorSubcoreMesh(core_axis_name="c", subcore_axis_name="s",
                                  num_cores=_NC, num_subcores=_NS)
    out_type = (
        jax.ShapeDtypeStruct((_B * _P * _S, _DP), jnp.float32),
        jax.ShapeDtypeStruct((_B * _P * _S,), jnp.int32),
    )
    scratch = [
        pltpu.VMEM((_N,), jnp.float32),
        pltpu.VMEM((_N,), jnp.float32),
        pltpu.VMEM((_N,), jnp.float32),
        pltpu.VMEM((_PC,), jnp.float32),
        pltpu.VMEM((_PC,), jnp.float32),
        pltpu.VMEM((_PC,), jnp.float32),
        pltpu.VMEM((_N + _L,), jnp.float32),
        pltpu.VMEM((_N + _L,), jnp.int32),
        pltpu.VMEM((_GR,), jnp.int32),
        pltpu.VMEM((_GR,), jnp.int32),
        pltpu.VMEM((_GR, _DP), jnp.float32),
        pltpu.SemaphoreType.DMA,
    ]
    f = pl.kernel(_sc_body, out_type=out_type, mesh=mesh, scratch_types=scratch,
                  compiler_params=pltpu.CompilerParams(
                      needs_layout_passes=False, use_tc_tiling_on_sc=False))
    nppad, idxf = f(cxt, xyzt, aug)
    nppad = nppad.reshape(_B, _P, _S, _DP)
    return (nppad[..., :_D], idxf.reshape(_B, _P, _S), nppad[..., :3])
